# Initial kernel scaffold; baseline (speedup 1.0000x reference)
import jax
import jax.numpy as jnp
from jax.experimental import pallas as pl


def kernel(x_nchw, wc1, bc1, wc2, bc2, wc3, bc3, wl1, bl1, wl2, bl2):
    raise NotImplementedError("write your pallas kernel here")


# speedup vs baseline: 1.6977x; 1.6977x over previous
import functools

import jax
import jax.numpy as jnp
from jax.experimental import pallas as pl
from jax.experimental.pallas import tpu as pltpu

_VMEM_LIMIT = 48 * 1024 * 1024


def _conv_body(x_ref, w_ref, b_ref, o_ref, *, H, wh, k3, co):
    rows = H * 2 * wh
    acc = None
    for dh in range(3):
        lhs = x_ref[0, dh:dh + H].reshape(rows, k3)
        part = jnp.dot(lhs, w_ref[dh], preferred_element_type=jnp.float32)
        acc = part if acc is None else acc + part
    y = jnp.maximum(acc + b_ref[...], 0.0)
    y = y.reshape(H // 2, 2, 2, wh, co)
    y = jnp.maximum(jnp.maximum(y[:, 0, 0], y[:, 0, 1]),
                    jnp.maximum(y[:, 1, 0], y[:, 1, 1]))
    o_ref[0] = y.astype(o_ref.dtype)


def _conv_pool(x, w3, b):
    B, H, W, C = x.shape
    Co = w3.shape[-1]
    k3 = 3 * C
    wh = W // 2
    xp = jnp.pad(x, ((0, 0), (1, 1), (1, 1), (0, 0)))
    cols = jnp.concatenate([xp[:, :, d:d + W, :] for d in range(3)], axis=-1)
    cols = cols.reshape(B, H + 2, wh, 2, k3).transpose(0, 1, 3, 2, 4)
    wk = w3.reshape(3, k3, Co)
    body = functools.partial(_conv_body, H=H, wh=wh, k3=k3, co=Co)
    return pl.pallas_call(
        body,
        out_shape=jax.ShapeDtypeStruct((B, H // 2, wh, Co), jnp.bfloat16),
        grid=(B,),
        in_specs=[
            pl.BlockSpec((1, H + 2, 2, wh, k3), lambda i: (i, 0, 0, 0, 0)),
            pl.BlockSpec((3, k3, Co), lambda i: (0, 0, 0)),
            pl.BlockSpec((1, Co), lambda i: (0, 0)),
        ],
        out_specs=pl.BlockSpec((1, H // 2, wh, Co), lambda i: (i, 0, 0, 0)),
        compiler_params=pltpu.CompilerParams(
            dimension_semantics=("parallel",),
            vmem_limit_bytes=_VMEM_LIMIT,
        ),
    )(cols, wk, b)


def _mlp_body(x_ref, w1_ref, b1_ref, w2_ref, b2_ref, o_ref):
    h = jnp.dot(x_ref[...], w1_ref[...], preferred_element_type=jnp.float32)
    h = jnp.maximum(h + b1_ref[...], 0.0)
    o_ref[...] = jnp.dot(h, w2_ref[...], preferred_element_type=jnp.float32) + b2_ref[...]


def _mlp(x, w1, b1, w2, b2):
    B, K = x.shape
    N1 = w1.shape[1]
    N2 = w2.shape[1]
    return pl.pallas_call(
        _mlp_body,
        out_shape=jax.ShapeDtypeStruct((B, N2), jnp.float32),
        grid=(1,),
        in_specs=[
            pl.BlockSpec((B, K), lambda i: (0, 0)),
            pl.BlockSpec((K, N1), lambda i: (0, 0)),
            pl.BlockSpec((1, N1), lambda i: (0, 0)),
            pl.BlockSpec((N1, N2), lambda i: (0, 0)),
            pl.BlockSpec((1, N2), lambda i: (0, 0)),
        ],
        out_specs=pl.BlockSpec((B, N2), lambda i: (0, 0)),
        compiler_params=pltpu.CompilerParams(
            dimension_semantics=("arbitrary",),
            vmem_limit_bytes=_VMEM_LIMIT,
        ),
    )(x, w1, b1, w2, b2)


def kernel(x_nchw, wc1, bc1, wc2, bc2, wc3, bc3, wl1, bl1, wl2, bl2):
    bf = jnp.bfloat16
    x = jnp.transpose(x_nchw, (0, 2, 3, 1)).astype(bf)
    x = _conv_pool(x, wc1.astype(bf), bc1)
    x = _conv_pool(x, wc2.astype(bf), bc2)
    x = _conv_pool(x, wc3.astype(bf), bc3)
    x = x.reshape(x.shape[0], -1)
    return _mlp(x, wl1.astype(bf), bl1, wl2.astype(jnp.float32), bl2)
